# baseline (speedup 1.0000x reference)
import functools

import jax
import jax.numpy as jnp
from jax import lax
from jax.experimental import pallas as pl
from jax.experimental.pallas import tpu as pltpu
from jax.experimental.pallas import tpu_sc as plsc

_LANES = 16


def _make_sc_add(B, T, D, V):
    info = plsc.get_sparse_core_info()
    NC, NS = info.num_cores, info.num_subcores
    NW = NC * NS
    t_per_w = T // NW
    CH = min(32, t_per_w)
    n_chunks = t_per_w // CH
    mesh = plsc.VectorSubcoreMesh(core_axis_name="c", subcore_axis_name="s")

    @functools.partial(
        pl.kernel,
        mesh=mesh,
        out_type=jax.ShapeDtypeStruct((B, T, D), jnp.float32),
        scratch_types=[
            pltpu.VMEM((CH,), jnp.int32),
            pltpu.VMEM((CH, D), jnp.float32),
            pltpu.VMEM((CH, D), jnp.float32),
            pltpu.SemaphoreType.DMA,
        ],
    )
    def sc_add(x_hbm, idx_hbm, pos_hbm, out_hbm, idx_v, pos_v, x_v, sem):
        wid = lax.axis_index("s") * NC + lax.axis_index("c")
        t_base = wid * t_per_w

        def chunk(ci, carry):
            t0 = t_base + ci * CH
            pltpu.sync_copy(idx_hbm.at[pl.ds(t0, CH)], idx_v)
            pltpu.async_copy(pos_hbm.at[idx_v], pos_v, sem).wait()
            for b in range(B):
                pltpu.sync_copy(x_hbm.at[b, pl.ds(t0, CH)], x_v)
                for r in range(CH):
                    def colgrp(j, c2):
                        sl = pl.ds(j * _LANES, _LANES)
                        x_v[r, sl] = x_v[r, sl] + pos_v[r, sl]
                        return c2
                    lax.fori_loop(0, D // _LANES, colgrp, 0)
                pltpu.sync_copy(x_v, out_hbm.at[b, pl.ds(t0, CH)])
            return carry

        lax.fori_loop(0, n_chunks, chunk, 0)

    return sc_add


def kernel(x, offset, pos_embedding):
    B, T, D = x.shape
    V = pos_embedding.shape[0]
    positions = jnp.clip(
        jnp.arange(T, dtype=jnp.int32) + jnp.asarray(offset, jnp.int32).astype(jnp.int32),
        0, V - 1)
    return _make_sc_add(B, T, D, V)(x, positions, pos_embedding)


# speedup vs baseline: 1.9975x; 1.9975x over previous
import functools

import jax
import jax.numpy as jnp
from jax import lax
from jax.experimental import pallas as pl
from jax.experimental.pallas import tpu as pltpu
from jax.experimental.pallas import tpu_sc as plsc

_LANES = 16


def _make_sc_add(B, T, D, V):
    info = plsc.get_sparse_core_info()
    NC, NS = info.num_cores, info.num_subcores
    NW = NC * NS
    t_per_w = T // NW
    CH = min(16, t_per_w)
    n_chunks = t_per_w // CH
    UNROLL = 8
    n_grp = D // _LANES
    mesh = plsc.VectorSubcoreMesh(core_axis_name="c", subcore_axis_name="s")

    @functools.partial(
        pl.kernel,
        mesh=mesh,
        out_type=jax.ShapeDtypeStruct((B, T, D), jnp.float32),
        scratch_types=[
            pltpu.VMEM((CH,), jnp.int32),
            pltpu.VMEM((CH, D), jnp.float32),
            pltpu.VMEM((B, CH, D), jnp.float32),
            pltpu.SemaphoreType.DMA,
        ],
    )
    def sc_add(x_hbm, idx_hbm, pos_hbm, out_hbm, idx_v, pos_v, x_v, sem):
        wid = lax.axis_index("s") * NC + lax.axis_index("c")
        t_base = wid * t_per_w

        def chunk(ci, carry):
            t0 = t_base + ci * CH
            pltpu.sync_copy(idx_hbm.at[pl.ds(t0, CH)], idx_v)
            pltpu.async_copy(pos_hbm.at[idx_v], pos_v, sem).wait()
            for b in range(B):
                pltpu.sync_copy(x_hbm.at[b, pl.ds(t0, CH)], x_v.at[b])
            for r in range(CH):
                def colgrp(jj, c2):
                    for u in range(UNROLL):
                        sl = pl.ds((jj * UNROLL + u) * _LANES, _LANES)
                        p = pos_v[r, sl]
                        for b in range(B):
                            x_v[b, r, sl] = x_v[b, r, sl] + p
                    return c2
                lax.fori_loop(0, n_grp // UNROLL, colgrp, 0)
            for b in range(B):
                pltpu.sync_copy(x_v.at[b], out_hbm.at[b, pl.ds(t0, CH)])
            return carry

        lax.fori_loop(0, n_chunks, chunk, 0)

    return sc_add


def kernel(x, offset, pos_embedding):
    B, T, D = x.shape
    V = pos_embedding.shape[0]
    positions = jnp.clip(
        jnp.arange(T, dtype=jnp.int32) + jnp.asarray(offset, jnp.int32).astype(jnp.int32),
        0, V - 1)
    return _make_sc_add(B, T, D, V)(x, positions, pos_embedding)


# speedup vs baseline: 3.5155x; 1.7599x over previous
import functools

import jax
import jax.numpy as jnp
from jax import lax
from jax.experimental import pallas as pl
from jax.experimental.pallas import tpu as pltpu
from jax.experimental.pallas import tpu_sc as plsc

_LANES = 16


def _make_sc_add(B, T, D, V):
    info = plsc.get_sparse_core_info()
    NC, NS = info.num_cores, info.num_subcores
    NW = NC * NS
    t_per_w = T // NW
    CH = min(8, t_per_w)
    n_chunks = t_per_w // CH
    UNROLL = 8
    n_grp = D // _LANES
    mesh = plsc.VectorSubcoreMesh(core_axis_name="c", subcore_axis_name="s")

    @functools.partial(
        pl.kernel,
        mesh=mesh,
        out_type=jax.ShapeDtypeStruct((B, T, D), jnp.float32),
        scratch_types=[
            pltpu.VMEM((t_per_w,), jnp.int32),
            pltpu.VMEM((2, CH, D), jnp.float32),
            pltpu.VMEM((2, B, CH, D), jnp.float32),
            pltpu.SemaphoreType.DMA((2,)),
            pltpu.SemaphoreType.DMA((2,)),
            pltpu.SemaphoreType.DMA((2,)),
        ],
    )
    def sc_add(x_hbm, idx_hbm, pos_hbm, out_hbm, idx_all, pos_v, x_v,
               sg, sx, so):
        wid = lax.axis_index("s") * NC + lax.axis_index("c")
        t_base = wid * t_per_w
        pltpu.sync_copy(idx_hbm.at[pl.ds(t_base, t_per_w)], idx_all)

        def in_copies(ci, p):
            t0 = t_base + ci * CH
            g = pltpu.make_async_copy(
                pos_hbm.at[idx_all.at[pl.ds(ci * CH, CH)]],
                pos_v.at[p], sg.at[p])
            xc = pltpu.make_async_copy(
                x_hbm.at[:, pl.ds(t0, CH)], x_v.at[p], sx.at[p])
            return g, xc

        def out_copy(ci, p):
            t0 = t_base + ci * CH
            return pltpu.make_async_copy(
                x_v.at[p], out_hbm.at[:, pl.ds(t0, CH)], so.at[p])

        for d in in_copies(0, 0):
            d.start()

        def chunk(ci_pair, carry):
            for p in range(2):
                ci = ci_pair * 2 + p
                @pl.when(ci + 1 < n_chunks)
                def _():
                    @pl.when(ci >= 1)
                    def _():
                        out_copy(ci - 1, 1 - p).wait()
                    for d in in_copies(ci + 1, 1 - p):
                        d.start()
                for d in in_copies(ci, p):
                    d.wait()
                for r in range(CH):
                    def colgrp(jj, c2):
                        for u in range(UNROLL):
                            sl = pl.ds((jj * UNROLL + u) * _LANES, _LANES)
                            pv = pos_v[p, r, sl]
                            for b in range(B):
                                x_v[p, b, r, sl] = x_v[p, b, r, sl] + pv
                        return c2
                    lax.fori_loop(0, n_grp // UNROLL, colgrp, 0)
                out_copy(ci, p).start()
            return carry

        lax.fori_loop(0, n_chunks // 2, chunk, 0)
        out_copy(n_chunks - 2, 0).wait()
        out_copy(n_chunks - 1, 1).wait()

    return sc_add


def kernel(x, offset, pos_embedding):
    B, T, D = x.shape
    V = pos_embedding.shape[0]
    positions = jnp.clip(
        jnp.arange(T, dtype=jnp.int32) + jnp.asarray(offset, jnp.int32).astype(jnp.int32),
        0, V - 1)
    return _make_sc_add(B, T, D, V)(x, positions, pos_embedding)
